# baseline (speedup 1.0000x reference)
import functools

import jax
import jax.numpy as jnp
from jax import lax
from jax.experimental import pallas as pl
from jax.experimental.pallas import tpu as pltpu
from jax.experimental.pallas import tpu_sc as plsc

NV = 10000
NE = 160000
D = 128
M = 320000
NB = 64
EPS = 1e-5
NW = 32
ROWS_V = 320
SUB_E = 632
NP_E = 8
CHF = 2000
PADC = 128


@functools.cache
def _mesh():
    return plsc.VectorSubcoreMesh(core_axis_name="c", subcore_axis_name="s")


def _wid():
    return lax.axis_index("s") * 2 + lax.axis_index("c")


def _i32(x):
    return jnp.asarray(x, jnp.int32)


def _make_filter(narr, nbuck, sub, flush):
    nlists = NW * nbuck
    cap = M + 3 * flush
    bufw = flush + CHF + 16
    outs = [jax.ShapeDtypeStruct((nlists * cap,), jnp.int32)
            for _ in range(narr)]
    outs += [jax.ShapeDtypeStruct((nlists * 16,), jnp.int32)] * 2
    scratch = ([pltpu.VMEM((CHF,), jnp.int32) for _ in range(narr)]
               + [pltpu.VMEM((bufw,), jnp.int32)
                  for _ in range(narr * nbuck)]
               + [pltpu.VMEM((16,), jnp.int32), pltpu.SemaphoreType.DMA])

    @functools.partial(pl.kernel, mesh=_mesh(), out_type=outs,
                       scratch_types=scratch,
                       compiler_params=pltpu.CompilerParams(
                           needs_layout_passes=False))
    def k(*args):
        arrs = args[:narr]
        list_outs = args[narr:2 * narr]
        cr_out, cp_out = args[2 * narr], args[2 * narr + 1]
        pos = 2 * narr + 2
        sts = args[pos:pos + narr]
        pos += narr
        bufs = [args[pos + b * narr:pos + (b + 1) * narr]
                for b in range(nbuck)]
        pos += narr * nbuck
        c16 = args[pos]

        w = _wid()

        def flush_bucket(b, cur, flushed):
            lid = w * nbuck + b

            def do_flush(args2):
                cur, flushed = args2
                for a in range(narr):
                    pltpu.sync_copy(
                        bufs[b][a].at[pl.ds(0, flush)],
                        list_outs[a].at[pl.ds(pl.multiple_of(lid * cap + flushed, 8), flush)])
                tail = cur - flush

                def mv(i, _):
                    for a in range(narr):
                        v = bufs[b][a][pl.ds(flush + i * 16, 16)]
                        bufs[b][a][pl.ds(i * 16, 16)] = v
                    return 0
                lax.fori_loop(0, (tail + 15) // 16, mv, 0)
                return cur - flush, flushed + flush

            return lax.cond(cur >= flush, do_flush, lambda a2: a2,
                            (cur, flushed))

        def blk_body(blk, carry):
            curs, flus = list(carry[0]), list(carry[1])
            for a in range(narr):
                pltpu.sync_copy(arrs[a].at[pl.ds(pl.multiple_of(blk * CHF, 8), CHF)], sts[a])

            def grp(g, cc):
                curs = list(cc)
                dv = sts[0][pl.ds(g * 16, 16)]
                vals = [sts[a][pl.ds(g * 16, 16)] for a in range(1, narr)]
                for b in range(nbuck):
                    lid = w * nbuck + b
                    lo = lid * sub
                    msk = (dv >= lo) & (dv < lo + sub)
                    inc = plsc.cumsum(msk.astype(jnp.int32))
                    pos = curs[b] + inc - 1
                    plsc.store_scatter(bufs[b][0], [pos], dv - lo, mask=msk)
                    for a in range(1, narr):
                        plsc.store_scatter(bufs[b][a], [pos],
                                           vals[a - 1], mask=msk)
                    curs[b] = curs[b] + inc[15]
                return tuple(curs)

            curs = list(lax.fori_loop(0, CHF // 16, grp, tuple(curs)))
            for b in range(nbuck):
                curs[b], flus[b] = flush_bucket(b, curs[b], flus[b])
            return tuple(curs), tuple(flus)

        zero = _i32(0)
        curs, flus = lax.fori_loop(
            0, M // CHF, blk_body,
            ((zero,) * nbuck, (zero,) * nbuck))

        for b in range(nbuck):
            lid = w * nbuck + b
            cur, flushed = curs[b], flus[b]
            n_real = flushed + cur
            n_pad = ((n_real + PADC - 1) // PADC) * PADC
            npad = n_pad - n_real

            def padl(i, c):
                pos = c + i * 16 + lax.iota(jnp.int32, 16)
                ones = lax.iota(jnp.int32, 16) >= 0
                plsc.store_scatter(bufs[b][0], [pos],
                                   jnp.full((16,), sub, jnp.int32),
                                   mask=ones)
                for a in range(1, narr):
                    plsc.store_scatter(bufs[b][a], [pos],
                                       jnp.zeros((16,), jnp.int32),
                                       mask=ones)
                return c
            lax.fori_loop(0, (npad + 15) // 16, padl, cur)
            cur = cur + npad

            def fin(i, carry):
                for a in range(narr):
                    pltpu.sync_copy(
                        bufs[b][a].at[pl.ds(i * flush, flush)],
                        list_outs[a].at[
                            pl.ds(pl.multiple_of(lid * cap + carry + i * flush, 8), flush)])
                return carry
            lax.fori_loop(0, (cur + flush - 1) // flush, fin, flushed)

            c16[...] = jnp.full((16,), n_real, jnp.int32)
            pltpu.sync_copy(c16, cr_out.at[pl.ds(pl.multiple_of(lid * 16, 8), 16)])
            c16[...] = jnp.full((16,), n_pad, jnp.int32)
            pltpu.sync_copy(c16, cp_out.at[pl.ds(pl.multiple_of(lid * 16, 8), 16)])

    return k, cap


def _make_deg(nbuck, sub, cap):
    acc_len = (sub + 1) * 16
    out_len = NW * nbuck * sub * 16

    @functools.partial(
        pl.kernel, mesh=_mesh(),
        out_type=jax.ShapeDtypeStruct((out_len,), jnp.float32),
        scratch_types=[pltpu.VMEM((acc_len,), jnp.float32),
                       pltpu.VMEM((PADC,), jnp.int32),
                       pltpu.VMEM((16,), jnp.int32),
                       pltpu.SemaphoreType.DMA],
        compiler_params=pltpu.CompilerParams(needs_layout_passes=False))
    def k(ld_hbm, cp_hbm, deg_out, acc, ldb, c16, sem):
        w = _wid()
        ones = jnp.ones((16,), jnp.float32)

        def bucket(b, _):
            lid = w * nbuck + b
            pltpu.sync_copy(cp_hbm.at[pl.ds(pl.multiple_of(lid * 16, 8), 16)], c16)
            n_pad = c16[...][0]

            def zr(i, _):
                acc[pl.ds(i * 16, 16)] = jnp.zeros((16,), jnp.float32)
                return 0
            lax.fori_loop(0, acc_len // 16, zr, 0)

            def chunk(j, _):
                pltpu.sync_copy(
                    ld_hbm.at[pl.ds(pl.multiple_of(lid * cap + j * PADC, 8), PADC)], ldb)

                def grp(g, _):
                    ldv = ldb[pl.ds(g * 16, 16)] * 16
                    for jj in range(16):
                        plsc.addupdate(acc.at[pl.ds(pl.multiple_of(ldv[jj], 16), 16)], ones)
                    return 0
                lax.fori_loop(0, PADC // 16, grp, 0)
                return 0
            lax.fori_loop(0, n_pad // PADC, chunk, 0)
            pltpu.sync_copy(acc.at[pl.ds(0, sub * 16)],
                            deg_out.at[pl.ds(pl.multiple_of(lid * sub * 16, 8), sub * 16)])
            return 0
        lax.fori_loop(0, nbuck, bucket, 0)
    return k


def _make_msgpass(nbuck, sub, cap, two_tables, relu, has_bias):
    acc_len = (sub + 1) * D
    s_len = NW * nbuck * sub * D
    outs = [jax.ShapeDtypeStruct((s_len,), jnp.float32),
            jax.ShapeDtypeStruct((NW * 256,), jnp.float32)]
    scratch = [pltpu.VMEM((acc_len,), jnp.float32),
               pltpu.VMEM((PADC,), jnp.int32),
               pltpu.VMEM((PADC,), jnp.int32)]
    if two_tables:
        scratch.append(pltpu.VMEM((PADC,), jnp.int32))
    scratch.append(pltpu.VMEM((PADC, D), jnp.float32))
    if two_tables:
        scratch.append(pltpu.VMEM((PADC, D), jnp.float32))
    scratch += [pltpu.VMEM((D,), jnp.float32),
                pltpu.VMEM((256,), jnp.float32),
                pltpu.VMEM((16,), jnp.int32),
                pltpu.SemaphoreType.DMA]

    @functools.partial(pl.kernel, mesh=_mesh(), out_type=outs,
                       scratch_types=scratch,
                       compiler_params=pltpu.CompilerParams(
                           needs_layout_passes=False))
    def k(*args):
        pos = 0
        p_hbm = args[pos]; pos += 1
        if two_tables:
            q_hbm = args[pos]; pos += 1
        bias_hbm = args[pos]; pos += 1
        ld_hbm, src_hbm = args[pos], args[pos + 1]; pos += 2
        if two_tables:
            qid_hbm = args[pos]; pos += 1
        cp_hbm = args[pos]; pos += 1
        s_out, st_out = args[pos], args[pos + 1]; pos += 2
        acc, ldb, srcb = args[pos], args[pos + 1], args[pos + 2]; pos += 3
        if two_tables:
            qidb = args[pos]; pos += 1
        prows = args[pos]; pos += 1
        if two_tables:
            qrows = args[pos]; pos += 1
        bv, stb, c16, sem = args[pos:pos + 4]

        w = _wid()
        if has_bias:
            pltpu.sync_copy(bias_hbm, bv)
            bias = [bv[pl.ds(kk * 16, 16)] for kk in range(8)]
        zero = jnp.zeros((16,), jnp.float32)

        def zst(i, _):
            stb[pl.ds(i * 16, 16)] = jnp.zeros((16,), jnp.float32)
            return 0
        lax.fori_loop(0, 16, zst, 0)

        def bucket(b, _):
            lid = w * nbuck + b
            pltpu.sync_copy(cp_hbm.at[pl.ds(pl.multiple_of(lid * 16, 8), 16)], c16)
            n_pad = c16[...][0]

            def zr(i, _):
                acc[pl.ds(i * 16, 16)] = jnp.zeros((16,), jnp.float32)
                return 0
            lax.fori_loop(0, acc_len // 16, zr, 0)

            def chunk(j, stats):
                off = lid * cap + j * PADC
                pltpu.sync_copy(ld_hbm.at[pl.ds(pl.multiple_of(off, 8), PADC)], ldb)
                pltpu.sync_copy(src_hbm.at[pl.ds(pl.multiple_of(off, 8), PADC)], srcb)
                cp1 = pltpu.async_copy(p_hbm.at[srcb], prows, sem)
                if two_tables:
                    pltpu.sync_copy(qid_hbm.at[pl.ds(pl.multiple_of(off, 8), PADC)], qidb)
                    cp2 = pltpu.async_copy(q_hbm.at[qidb], qrows, sem)
                cp1.wait()
                if two_tables:
                    cp2.wait()

                def grp(g, stats):
                    st = list(stats)
                    ldv = ldb[pl.ds(g * 16, 16)] * D
                    for jj in range(16):
                        base = ldv[jj]
                        mm = g * 16 + jj
                        for kk in range(8):
                            t = prows[mm, pl.ds(kk * 16, 16)]
                            if two_tables:
                                t = t + qrows[mm, pl.ds(kk * 16, 16)]
                            if has_bias:
                                t = t + bias[kk]
                            if relu:
                                t = jnp.maximum(t, 0.0)
                            plsc.addupdate(
                                acc.at[pl.ds(pl.multiple_of(base + kk * 16, 16), 16)], t)
                            st[kk] = st[kk] + t
                            st[8 + kk] = st[8 + kk] + t * t
                    return tuple(st)
                return lax.fori_loop(0, PADC // 16, grp, stats)

            stats = lax.fori_loop(0, n_pad // PADC, chunk, (zero,) * 16)
            for kk in range(16):
                stb[pl.ds(kk * 16, 16)] = (stb[pl.ds(kk * 16, 16)]
                                           + stats[kk])
            pltpu.sync_copy(acc.at[pl.ds(0, sub * D)],
                            s_out.at[pl.ds(pl.multiple_of(lid * sub * D, 8), sub * D)])
            return 0
        lax.fori_loop(0, nbuck, bucket, 0)
        pltpu.sync_copy(stb, st_out.at[pl.ds(pl.multiple_of(w * 256, 8), 256)])
    return k


def _tables_tc(x, w_all, b_all, relu_tail):
    R = x.shape[0]
    K = w_all.shape[1]
    BR = 1024
    nout = K // D

    def body(x_ref, w_ref, b_ref, *o_refs):
        acc = jnp.dot(x_ref[...], w_ref[...],
                      preferred_element_type=jnp.float32) + b_ref[...]
        if relu_tail:
            keep = acc[:, :K - relu_tail]
            r = jnp.maximum(acc[:, K - relu_tail:], 0.0)
            acc = jnp.concatenate([keep, r], axis=1)
        for i in range(nout):
            o_refs[i][...] = acc[:, i * D:(i + 1) * D]

    outs = [jax.ShapeDtypeStruct((R, D), jnp.float32) for _ in range(nout)]
    out_specs = [pl.BlockSpec((BR, D), lambda i: (i, 0))
                 for _ in range(nout)]
    return pl.pallas_call(
        body,
        grid=(pl.cdiv(R, BR),),
        in_specs=[pl.BlockSpec((BR, D), lambda i: (i, 0)),
                  pl.BlockSpec((D, K), lambda i: (0, 0)),
                  pl.BlockSpec((1, K), lambda i: (0, 0))],
        out_specs=out_specs,
        out_shape=outs,
    )(x, w_all, b_all.reshape(1, K))


def _update_tc(xprev, a, c, aggs, w1, b1, w2, b2):
    R = xprev.shape[0]
    BR = 1024
    nagg = len(aggs)
    grid = pl.cdiv(R, BR)

    def body(*refs):
        x_ref = refs[0]
        a_ref, c_ref = refs[1], refs[2]
        pos = 3
        inp = x_ref[...] * a_ref[...] + c_ref[...]
        for _ in range(nagg):
            s_ref, deg_ref, sc_ref, sh_ref = refs[pos:pos + 4]
            pos += 4
            deg = deg_ref[...][:, 0:1]
            inp = inp + s_ref[...] * sc_ref[...] + deg * sh_ref[...]
        w1_ref, b1_ref, w2_ref, b2_ref = refs[pos:pos + 4]
        o_ref, ps_ref = refs[pos + 4], refs[pos + 5]
        h = jnp.maximum(jnp.dot(inp, w1_ref[...],
                                preferred_element_type=jnp.float32)
                        + b1_ref[...], 0.0)
        h = jnp.maximum(jnp.dot(h, w2_ref[...],
                                preferred_element_type=jnp.float32)
                        + b2_ref[...], 0.0)
        o_ref[...] = h
        row = (jax.lax.broadcasted_iota(jnp.int32, (BR, 1), 0)
               + pl.program_id(0) * BR)
        hm = jnp.where(row < R, h, 0.0)
        ps_ref[...] = jnp.concatenate(
            [jnp.sum(hm, axis=0)[None], jnp.sum(hm * hm, axis=0)[None],
             jnp.zeros((6, D), jnp.float32)], axis=0)[None]

    in_specs = [pl.BlockSpec((BR, D), lambda i: (i, 0)),
                pl.BlockSpec((1, D), lambda i: (0, 0)),
                pl.BlockSpec((1, D), lambda i: (0, 0))]
    args = [xprev, a.reshape(1, D), c.reshape(1, D)]
    for (s, deg, sc, sh) in aggs:
        in_specs.append(pl.BlockSpec((BR, D), lambda i: (i, 0)))
        args.append(s)
        in_specs.append(pl.BlockSpec((BR, 16), lambda i: (i, 0)))
        args.append(deg)
        in_specs.append(pl.BlockSpec((1, D), lambda i: (0, 0)))
        args.append(sc.reshape(1, D))
        in_specs.append(pl.BlockSpec((1, D), lambda i: (0, 0)))
        args.append(sh.reshape(1, D))
    in_specs += [pl.BlockSpec((D, D), lambda i: (0, 0)),
                 pl.BlockSpec((1, D), lambda i: (0, 0)),
                 pl.BlockSpec((D, D), lambda i: (0, 0)),
                 pl.BlockSpec((1, D), lambda i: (0, 0))]
    args += [w1, b1.reshape(1, D), w2, b2.reshape(1, D)]
    outs = [jax.ShapeDtypeStruct((R, D), jnp.float32),
            jax.ShapeDtypeStruct((grid, 8, D), jnp.float32)]
    out_specs = [pl.BlockSpec((BR, D), lambda i: (i, 0)),
                 pl.BlockSpec((1, 8, D), lambda i: (i, 0, 0))]
    return pl.pallas_call(body, grid=(grid,), in_specs=in_specs,
                          out_specs=out_specs, out_shape=outs)(*args)


def _pool_tc(x, a, c, batch3, R, BRP):
    grid = R // BRP

    def body(b_ref, x_ref, a_ref, c_ref, o_ref, cnt_ref):
        i = pl.program_id(0)

        @pl.when(i == 0)
        def _():
            o_ref[...] = jnp.zeros_like(o_ref)
            cnt_ref[...] = jnp.zeros_like(cnt_ref)
        seg = jax.lax.broadcasted_iota(jnp.int32, (NB, BRP), 0)
        onehot = (b_ref[0, 0, :][None, :] == seg).astype(jnp.float32)
        xb = x_ref[...] * a_ref[...] + c_ref[...]
        o_ref[...] += jnp.dot(onehot, xb, preferred_element_type=jnp.float32)
        cnt_ref[...] += jnp.dot(onehot, jnp.ones((BRP, D), jnp.float32),
                                preferred_element_type=jnp.float32)

    return pl.pallas_call(
        body, grid=(grid,),
        in_specs=[pl.BlockSpec((1, 1, BRP), lambda i: (i, 0, 0)),
                  pl.BlockSpec((BRP, D), lambda i: (i, 0)),
                  pl.BlockSpec((1, D), lambda i: (0, 0)),
                  pl.BlockSpec((1, D), lambda i: (0, 0))],
        out_specs=[pl.BlockSpec((NB, D), lambda i: (0, 0)),
                   pl.BlockSpec((NB, D), lambda i: (0, 0))],
        out_shape=[jax.ShapeDtypeStruct((NB, D), jnp.float32),
                   jax.ShapeDtypeStruct((NB, D), jnp.float32)],
    )(batch3, x, a.reshape(1, D), c.reshape(1, D))


def _head_tc(pv, cv, pe, ce, w1, b1, w2, b2):
    CP = 128

    def body(pv_ref, cv_ref, pe_ref, ce_ref, w1_ref, b1_ref, w2_ref,
             b2_ref, o_ref):
        x = pv_ref[...] / cv_ref[...] + pe_ref[...] / ce_ref[...]
        h = jnp.maximum(jnp.dot(x, w1_ref[...],
                                preferred_element_type=jnp.float32)
                        + b1_ref[...], 0.0)
        o_ref[...] = jnp.dot(h, w2_ref[...],
                             preferred_element_type=jnp.float32) + b2_ref[...]

    return pl.pallas_call(
        body, grid=(1,),
        in_specs=[pl.BlockSpec((NB, D), lambda i: (0, 0))] * 4
        + [pl.BlockSpec((D, D), lambda i: (0, 0)),
           pl.BlockSpec((1, D), lambda i: (0, 0)),
           pl.BlockSpec((D, CP), lambda i: (0, 0)),
           pl.BlockSpec((1, CP), lambda i: (0, 0))],
        out_specs=pl.BlockSpec((NB, CP), lambda i: (0, 0)),
        out_shape=jax.ShapeDtypeStruct((NB, CP), jnp.float32),
    )(pv, cv, pe, ce, w1, b1, w2, b2)


_make_filter = functools.cache(_make_filter)
_make_deg = functools.cache(_make_deg)
_make_msgpass = functools.cache(_make_msgpass)

FLUSH_V = 2048
FLUSH_E = 1024


def _msg_stats(st, cr, cp, t_dummy):
    st = st.reshape(NW, 16, 16).astype(jnp.float32)
    s_sum = jnp.sum(st[:, 0:8], axis=0).reshape(D)
    s_sq = jnp.sum(st[:, 8:16], axis=0).reshape(D)
    n_dummy = jnp.sum((cp - cr).reshape(-1, 16)[:, 0]).astype(jnp.float32)
    s_sum = s_sum - n_dummy * t_dummy
    s_sq = s_sq - n_dummy * t_dummy * t_dummy
    mean = s_sum / M
    var = s_sq / M - mean * mean
    return mean, var


def _affine(mean, var, g, be):
    sc = g * jax.lax.rsqrt(var + EPS)
    return sc, be - mean * sc


def kernel(x_v, x_e, v_up_index, v_up_edge, e_down_index, e_down_vert,
           e_up_index, batch_v, batch_e, params):
    p = params
    filt_v, cap_v = _make_filter(3, 1, ROWS_V, FLUSH_V)
    filt_e3, cap_e = _make_filter(3, NP_E, SUB_E, FLUSH_E)
    filt_e2, _ = _make_filter(2, NP_E, SUB_E, FLUSH_E)

    vL, vS, vQ, vCR, vCP = filt_v(v_up_index[1], v_up_index[0], v_up_edge)
    dL, dS, dQ, dCR, dCP = filt_e3(e_down_index[1], e_down_index[0],
                                   e_down_vert)
    uL, uS, uCR, uCP = filt_e2(e_up_index[1], e_up_index[0])
    deg_v = _make_deg(1, ROWS_V, cap_v)(vL, vCP)
    deg_v = deg_v.reshape(NW * ROWS_V, 16)[:NV]
    deg_d = _make_deg(NP_E, SUB_E, cap_e)(dL, dCP)
    deg_d = deg_d.reshape(NW * NP_E * SUB_E, 16)[:NE]
    deg_u = _make_deg(NP_E, SUB_E, cap_e)(uL, uCP)
    deg_u = deg_u.reshape(NW * NP_E * SUB_E, 16)[:NE]

    vup_pass = _make_msgpass(1, ROWS_V, cap_v, True, True, True)
    ed_pass = _make_msgpass(NP_E, SUB_E, cap_e, True, True, True)
    eu_pass = _make_msgpass(NP_E, SUB_E, cap_e, False, False, False)

    hv, he = x_v, x_e
    a_v = jnp.ones((D,), jnp.float32)
    c_v = jnp.zeros((D,), jnp.float32)
    a_e = jnp.ones((D,), jnp.float32)
    c_e = jnp.zeros((D,), jnp.float32)

    for l in range(3):
        wv = p[f"L{l}_vup_W"]
        we = p[f"L{l}_edown_W"]
        wu = p[f"L{l}_eup_W"]
        wv_cat = jnp.concatenate([wv[:D], we[D:]], axis=1)
        wv_f = a_v[:, None] * wv_cat
        bv_f = c_v @ wv_cat
        Pv, Qv = _tables_tc(hv, wv_f, bv_f, 0)
        we_cat = jnp.concatenate([wv[D:], we[:D], wu], axis=1)
        we_f = a_e[:, None] * we_cat
        be_f = c_e @ we_cat
        be_f = be_f.at[2 * D:].add(p[f"L{l}_eup_b"])
        Qe, Pe, Ru = _tables_tc(he, we_f, be_f, D)

        sv_flat, v_st = vup_pass(Pv, Qe, p[f"L{l}_vup_b"], vL, vS, vQ, vCP)
        Sv = sv_flat.reshape(NW * ROWS_V, D)[:NV]
        se_flat, d_st = ed_pass(Pe, Qv, p[f"L{l}_edown_b"], dL, dS, dQ, dCP)
        Se = se_flat.reshape(NW * NP_E * SUB_E, D)[:NE]
        su_flat, u_st = eu_pass(Ru, p[f"L{l}_eup_b"], uL, uS, uCP)
        Su = su_flat.reshape(NW * NP_E * SUB_E, D)[:NE]

        td_v = jnp.maximum(Pv[0] + Qe[0] + p[f"L{l}_vup_b"], 0.0)
        mean, var = _msg_stats(v_st, vCR, vCP, td_v)
        sc_v, sh_v = _affine(mean, var, p[f"L{l}_vup_g"], p[f"L{l}_vup_be"])
        td_d = jnp.maximum(Pe[0] + Qv[0] + p[f"L{l}_edown_b"], 0.0)
        mean, var = _msg_stats(d_st, dCR, dCP, td_d)
        sc_d, sh_d = _affine(mean, var, p[f"L{l}_edown_g"],
                             p[f"L{l}_edown_be"])
        td_u = Ru[0]
        mean, var = _msg_stats(u_st, uCR, uCP, td_u)
        sc_u, sh_u = _affine(mean, var, p[f"L{l}_eup_g"], p[f"L{l}_eup_be"])

        hv, v_ps = _update_tc(hv, a_v, c_v,
                              [(Sv, deg_v, sc_v, sh_v)],
                              p[f"L{l}_vupd_W1"], p[f"L{l}_vupd_b1"],
                              p[f"L{l}_vupd_W2"], p[f"L{l}_vupd_b2"])
        he, e_ps = _update_tc(he, a_e, c_e,
                              [(Se, deg_d, sc_d, sh_d),
                               (Su, deg_u, sc_u, sh_u)],
                              p[f"L{l}_eupd_W1"], p[f"L{l}_eupd_b1"],
                              p[f"L{l}_eupd_W2"], p[f"L{l}_eupd_b2"])
        s1 = jnp.sum(v_ps[:, 0], axis=0)
        s2 = jnp.sum(v_ps[:, 1], axis=0)
        mean = s1 / NV
        var = s2 / NV - mean * mean
        a_v, c_v = _affine(mean, var, p[f"L{l}_vupd_g"], p[f"L{l}_vupd_be"])
        s1 = jnp.sum(e_ps[:, 0], axis=0)
        s2 = jnp.sum(e_ps[:, 1], axis=0)
        mean = s1 / NE
        var = s2 / NE - mean * mean
        a_e, c_e = _affine(mean, var, p[f"L{l}_eupd_g"], p[f"L{l}_eupd_be"])

    bv3 = batch_v.reshape(5, 1, 2000)
    be3 = batch_e.reshape(80, 1, 2000)
    pv, cv = _pool_tc(hv, a_v, c_v, bv3, NV, 2000)
    pe, ce = _pool_tc(he, a_e, c_e, be3, NE, 2000)
    cv = jnp.maximum(cv, 1.0)
    ce = jnp.maximum(ce, 1.0)
    w2p = jnp.zeros((D, 128), jnp.float32).at[:, :10].set(p["lin2_W"])
    b2p = jnp.zeros((128,), jnp.float32).at[:10].set(p["lin2_b"])
    out = _head_tc(pv, cv, pe, ce, p["lin1_W"], p["lin1_b"].reshape(1, D),
                   w2p, b2p.reshape(1, 128))
    return out[:, :10]


# speedup vs baseline: 1.0883x; 1.0883x over previous
import functools

import jax
import jax.numpy as jnp
from jax import lax
from jax.experimental import pallas as pl
from jax.experimental.pallas import tpu as pltpu
from jax.experimental.pallas import tpu_sc as plsc

NV = 10000
NE = 160000
D = 128
M = 320000
NB = 64
EPS = 1e-5
NW = 32
ROWS_V = 320
NP_V = 1
SUB_E = 500
NP_E = 10
GC = 64
SUP = 8
CHF = 2000
PADC = 128


@functools.cache
def _mesh():
    return plsc.VectorSubcoreMesh(core_axis_name="c", subcore_axis_name="s")


def _wid():
    return lax.axis_index("s") * 2 + lax.axis_index("c")


def _i32(x):
    return jnp.asarray(x, jnp.int32)


def _make_filter(narr, nbuck, sub, flush):
    nlists = NW * nbuck
    cap = M + 3 * flush
    bufw = flush + CHF + 16
    outs = [jax.ShapeDtypeStruct((nlists * cap,), jnp.int32)
            for _ in range(narr)]
    outs += [jax.ShapeDtypeStruct((nlists * 16,), jnp.int32)] * 2
    scratch = ([pltpu.VMEM((CHF,), jnp.int32) for _ in range(narr)]
               + [pltpu.VMEM((bufw,), jnp.int32)
                  for _ in range(narr * nbuck)]
               + [pltpu.VMEM((16,), jnp.int32), pltpu.SemaphoreType.DMA])

    @functools.partial(pl.kernel, mesh=_mesh(), out_type=outs,
                       scratch_types=scratch,
                       compiler_params=pltpu.CompilerParams(
                           needs_layout_passes=False))
    def k(*args):
        arrs = args[:narr]
        list_outs = args[narr:2 * narr]
        cr_out, cp_out = args[2 * narr], args[2 * narr + 1]
        pos = 2 * narr + 2
        sts = args[pos:pos + narr]
        pos += narr
        bufs = [args[pos + b * narr:pos + (b + 1) * narr]
                for b in range(nbuck)]
        pos += narr * nbuck
        c16 = args[pos]

        w = _wid()

        def flush_bucket(b, cur, flushed):
            lid = w * nbuck + b

            def do_flush(args2):
                cur, flushed = args2
                for a in range(narr):
                    pltpu.sync_copy(
                        bufs[b][a].at[pl.ds(0, flush)],
                        list_outs[a].at[pl.ds(pl.multiple_of(lid * cap + flushed, 8), flush)])
                tail = cur - flush

                def mv(i, _):
                    for a in range(narr):
                        v = bufs[b][a][pl.ds(flush + i * 16, 16)]
                        bufs[b][a][pl.ds(i * 16, 16)] = v
                    return 0
                lax.fori_loop(0, (tail + 15) // 16, mv, 0)
                return cur - flush, flushed + flush

            return lax.cond(cur >= flush, do_flush, lambda a2: a2,
                            (cur, flushed))

        def blk_body(blk, carry):
            curs, flus = list(carry[0]), list(carry[1])
            for a in range(narr):
                pltpu.sync_copy(arrs[a].at[pl.ds(pl.multiple_of(blk * CHF, 8), CHF)], sts[a])

            def grp(g, cc):
                curs = list(cc)
                dv = sts[0][pl.ds(g * 16, 16)]
                vals = [sts[a][pl.ds(g * 16, 16)] for a in range(1, narr)]
                for b in range(nbuck):
                    lid = w * nbuck + b
                    lo = lid * sub
                    msk = (dv >= lo) & (dv < lo + sub)
                    inc = plsc.cumsum(msk.astype(jnp.int32))
                    pos = curs[b] + inc - 1
                    plsc.store_scatter(bufs[b][0], [pos], dv - lo, mask=msk)
                    for a in range(1, narr):
                        plsc.store_scatter(bufs[b][a], [pos],
                                           vals[a - 1], mask=msk)
                    curs[b] = curs[b] + inc[15]
                return tuple(curs)

            curs = list(lax.fori_loop(0, CHF // 16, grp, tuple(curs)))
            for b in range(nbuck):
                curs[b], flus[b] = flush_bucket(b, curs[b], flus[b])
            return tuple(curs), tuple(flus)

        zero = _i32(0)
        curs, flus = lax.fori_loop(
            0, M // CHF, blk_body,
            ((zero,) * nbuck, (zero,) * nbuck))

        for b in range(nbuck):
            lid = w * nbuck + b
            cur, flushed = curs[b], flus[b]
            n_real = flushed + cur
            n_pad = ((n_real + PADC - 1) // PADC) * PADC
            npad = n_pad - n_real

            def padl(i, c):
                pos = c + i * 16 + lax.iota(jnp.int32, 16)
                ones = lax.iota(jnp.int32, 16) >= 0
                plsc.store_scatter(bufs[b][0], [pos],
                                   jnp.full((16,), sub, jnp.int32),
                                   mask=ones)
                for a in range(1, narr):
                    plsc.store_scatter(bufs[b][a], [pos],
                                       jnp.zeros((16,), jnp.int32),
                                       mask=ones)
                return c
            lax.fori_loop(0, (npad + 15) // 16, padl, cur)
            cur = cur + npad

            def fin(i, carry):
                for a in range(narr):
                    pltpu.sync_copy(
                        bufs[b][a].at[pl.ds(i * flush, flush)],
                        list_outs[a].at[
                            pl.ds(pl.multiple_of(lid * cap + carry + i * flush, 8), flush)])
                return carry
            lax.fori_loop(0, (cur + flush - 1) // flush, fin, flushed)

            c16[...] = jnp.full((16,), n_real, jnp.int32)
            pltpu.sync_copy(c16, cr_out.at[pl.ds(pl.multiple_of(lid * 16, 8), 16)])
            c16[...] = jnp.full((16,), n_pad, jnp.int32)
            pltpu.sync_copy(c16, cp_out.at[pl.ds(pl.multiple_of(lid * 16, 8), 16)])

    return k, cap


def _make_deg(nbuck, sub, cap):
    acc_len = (sub + 1) * 16
    out_len = NW * nbuck * sub * 16

    @functools.partial(
        pl.kernel, mesh=_mesh(),
        out_type=jax.ShapeDtypeStruct((out_len,), jnp.float32),
        scratch_types=[pltpu.VMEM((acc_len,), jnp.float32),
                       pltpu.VMEM((PADC,), jnp.int32),
                       pltpu.VMEM((16,), jnp.int32),
                       pltpu.SemaphoreType.DMA],
        compiler_params=pltpu.CompilerParams(needs_layout_passes=False))
    def k(ld_hbm, cp_hbm, deg_out, acc, ldb, c16, sem):
        w = _wid()
        ones = jnp.ones((16,), jnp.float32)

        def bucket(b, _):
            lid = w * nbuck + b
            pltpu.sync_copy(cp_hbm.at[pl.ds(pl.multiple_of(lid * 16, 8), 16)], c16)
            n_pad = c16[...][0]

            def zr(i, _):
                acc[pl.ds(i * 16, 16)] = jnp.zeros((16,), jnp.float32)
                return 0
            lax.fori_loop(0, acc_len // 16, zr, 0)

            def chunk(j, _):
                pltpu.sync_copy(
                    ld_hbm.at[pl.ds(pl.multiple_of(lid * cap + j * PADC, 8), PADC)], ldb)

                def grp(g, _):
                    ldv = ldb[pl.ds(g * 16, 16)] * 16
                    for jj in range(16):
                        plsc.addupdate(acc.at[pl.ds(pl.multiple_of(ldv[jj], 16), 16)], ones)
                    return 0
                lax.fori_loop(0, PADC // 16, grp, 0)
                return 0
            lax.fori_loop(0, n_pad // PADC, chunk, 0)
            pltpu.sync_copy(acc.at[pl.ds(0, sub * 16)],
                            deg_out.at[pl.ds(pl.multiple_of(lid * sub * 16, 8), sub * 16)])
            return 0
        lax.fori_loop(0, nbuck, bucket, 0)
    return k


def _make_msgpass(nbuck, sub, cap, two_tables, relu, has_bias):
    acc_len = (sub + 1) * D
    s_len = NW * nbuck * sub * D
    IW = SUP * GC
    outs = [jax.ShapeDtypeStruct((s_len,), jnp.float32),
            jax.ShapeDtypeStruct((NW * 256,), jnp.float32)]
    scratch = [pltpu.VMEM((acc_len,), jnp.float32),
               pltpu.VMEM((IW,), jnp.int32),
               pltpu.VMEM((IW,), jnp.int32)]
    if two_tables:
        scratch.append(pltpu.VMEM((IW,), jnp.int32))
    scratch.append(pltpu.VMEM((2 * GC, D), jnp.float32))
    if two_tables:
        scratch.append(pltpu.VMEM((2 * GC, D), jnp.float32))
    scratch += [pltpu.VMEM((D,), jnp.float32),
                pltpu.VMEM((256,), jnp.float32),
                pltpu.VMEM((16,), jnp.int32),
                pltpu.SemaphoreType.DMA,
                pltpu.SemaphoreType.DMA]

    @functools.partial(pl.kernel, mesh=_mesh(), out_type=outs,
                       scratch_types=scratch,
                       compiler_params=pltpu.CompilerParams(
                           needs_layout_passes=False))
    def k(*args):
        pos = 0
        p_hbm = args[pos]; pos += 1
        if two_tables:
            q_hbm = args[pos]; pos += 1
        bias_hbm = args[pos]; pos += 1
        ld_hbm, src_hbm = args[pos], args[pos + 1]; pos += 2
        if two_tables:
            qid_hbm = args[pos]; pos += 1
        cp_hbm = args[pos]; pos += 1
        s_out, st_out = args[pos], args[pos + 1]; pos += 2
        acc, ldb, srcb = args[pos], args[pos + 1], args[pos + 2]; pos += 3
        if two_tables:
            qidb = args[pos]; pos += 1
        prows = args[pos]; pos += 1
        if two_tables:
            qrows = args[pos]; pos += 1
        bv, stb, c16, sem0, sem1 = args[pos:pos + 5]
        gsems = (sem0, sem1)

        w = _wid()
        if has_bias:
            pltpu.sync_copy(bias_hbm, bv)
            bias = [bv[pl.ds(kk * 16, 16)] for kk in range(8)]
        zero = jnp.zeros((16,), jnp.float32)

        def zst(i, _):
            stb[pl.ds(i * 16, 16)] = jnp.zeros((16,), jnp.float32)
            return 0
        lax.fori_loop(0, 16, zst, 0)

        def bucket(b, _):
            lid = w * nbuck + b
            pltpu.sync_copy(cp_hbm.at[pl.ds(pl.multiple_of(lid * 16, 8), 16)], c16)
            n_pad = c16[...][0]
            nch = n_pad // GC

            def zr(i, _):
                acc[pl.ds(i * 16, 16)] = jnp.zeros((16,), jnp.float32)
                return 0
            lax.fori_loop(0, acc_len // 16, zr, 0)

            def g_copies(j):
                slot = j % 2
                idx = srcb.at[pl.ds(j * GC, GC)]
                dst = prows.at[pl.ds(slot * GC, GC)]
                cps = [pltpu.make_async_copy(p_hbm.at[idx], dst,
                                             gsems[slot])]
                if two_tables:
                    qidx = qidb.at[pl.ds(j * GC, GC)]
                    qdst = qrows.at[pl.ds(slot * GC, GC)]
                    cps.append(pltpu.make_async_copy(q_hbm.at[qidx], qdst,
                                                     gsems[slot]))
                return cps

            def compute(j, stats):
                slot = j % 2

                def grp(g, st_):
                    st = list(st_)
                    ldv = ldb[pl.ds(j * GC + g * 16, 16)] * D
                    for jj in range(16):
                        base = ldv[jj]
                        mm = slot * GC + g * 16 + jj
                        for kk in range(8):
                            t = prows[mm, pl.ds(kk * 16, 16)]
                            if two_tables:
                                t = t + qrows[mm, pl.ds(kk * 16, 16)]
                            if has_bias:
                                t = t + bias[kk]
                            if relu:
                                t = jnp.maximum(t, 0.0)
                            plsc.addupdate(
                                acc.at[pl.ds(pl.multiple_of(base + kk * 16, 16), 16)], t)
                            st[kk] = st[kk] + t
                            st[8 + kk] = st[8 + kk] + t * t
                    return tuple(st)
                return lax.fori_loop(0, GC // 16, grp, stats)

            def sup_body(s, stats):
                off = lid * cap + s * IW
                pltpu.sync_copy(ld_hbm.at[pl.ds(pl.multiple_of(off, 8), IW)],
                                ldb)
                pltpu.sync_copy(src_hbm.at[pl.ds(pl.multiple_of(off, 8), IW)],
                                srcb)
                if two_tables:
                    pltpu.sync_copy(
                        qid_hbm.at[pl.ds(pl.multiple_of(off, 8), IW)], qidb)
                for cp in g_copies(0):
                    cp.start()
                for j in range(SUP):
                    c = s * SUP + j
                    if j == 0:
                        for cp in g_copies(0):
                            cp.wait()
                    else:
                        @pl.when(c < nch)
                        def _(j=j):
                            for cp in g_copies(j):
                                cp.wait()
                    if j < SUP - 1:
                        @pl.when(c + 1 < nch)
                        def _(j=j):
                            for cp in g_copies(j + 1):
                                cp.start()
                    stats = lax.cond(c < nch,
                                     functools.partial(compute, j),
                                     lambda st: st, stats)
                return stats

            nsup = (nch + SUP - 1) // SUP
            stats = lax.fori_loop(0, nsup, sup_body, (zero,) * 16)
            for kk in range(16):
                stb[pl.ds(kk * 16, 16)] = (stb[pl.ds(kk * 16, 16)]
                                           + stats[kk])
            pltpu.sync_copy(acc.at[pl.ds(0, sub * D)],
                            s_out.at[pl.ds(pl.multiple_of(lid * sub * D, 8), sub * D)])
            return 0
        lax.fori_loop(0, nbuck, bucket, 0)
        pltpu.sync_copy(stb, st_out.at[pl.ds(pl.multiple_of(w * 256, 8), 256)])
    return k


def _tables_tc(x, w_all, b_all, relu_tail):
    R = x.shape[0]
    K = w_all.shape[1]
    BR = 1024
    nout = K // D

    def body(x_ref, w_ref, b_ref, *o_refs):
        acc = jnp.dot(x_ref[...], w_ref[...],
                      preferred_element_type=jnp.float32) + b_ref[...]
        if relu_tail:
            keep = acc[:, :K - relu_tail]
            r = jnp.maximum(acc[:, K - relu_tail:], 0.0)
            acc = jnp.concatenate([keep, r], axis=1)
        for i in range(nout):
            o_refs[i][...] = acc[:, i * D:(i + 1) * D]

    outs = [jax.ShapeDtypeStruct((R, D), jnp.float32) for _ in range(nout)]
    out_specs = [pl.BlockSpec((BR, D), lambda i: (i, 0))
                 for _ in range(nout)]
    return pl.pallas_call(
        body,
        grid=(pl.cdiv(R, BR),),
        in_specs=[pl.BlockSpec((BR, D), lambda i: (i, 0)),
                  pl.BlockSpec((D, K), lambda i: (0, 0)),
                  pl.BlockSpec((1, K), lambda i: (0, 0))],
        out_specs=out_specs,
        out_shape=outs,
    )(x, w_all, b_all.reshape(1, K))


def _update_tc(xprev, a, c, aggs, w1, b1, w2, b2):
    R = xprev.shape[0]
    BR = 1024
    nagg = len(aggs)
    grid = pl.cdiv(R, BR)

    def body(*refs):
        x_ref = refs[0]
        a_ref, c_ref = refs[1], refs[2]
        pos = 3
        inp = x_ref[...] * a_ref[...] + c_ref[...]
        for _ in range(nagg):
            s_ref, deg_ref, sc_ref, sh_ref = refs[pos:pos + 4]
            pos += 4
            deg = deg_ref[...][:, 0:1]
            inp = inp + s_ref[...] * sc_ref[...] + deg * sh_ref[...]
        w1_ref, b1_ref, w2_ref, b2_ref = refs[pos:pos + 4]
        o_ref, ps_ref = refs[pos + 4], refs[pos + 5]
        h = jnp.maximum(jnp.dot(inp, w1_ref[...],
                                preferred_element_type=jnp.float32)
                        + b1_ref[...], 0.0)
        h = jnp.maximum(jnp.dot(h, w2_ref[...],
                                preferred_element_type=jnp.float32)
                        + b2_ref[...], 0.0)
        o_ref[...] = h
        row = (jax.lax.broadcasted_iota(jnp.int32, (BR, 1), 0)
               + pl.program_id(0) * BR)
        hm = jnp.where(row < R, h, 0.0)
        ps_ref[...] = jnp.concatenate(
            [jnp.sum(hm, axis=0)[None], jnp.sum(hm * hm, axis=0)[None],
             jnp.zeros((6, D), jnp.float32)], axis=0)[None]

    in_specs = [pl.BlockSpec((BR, D), lambda i: (i, 0)),
                pl.BlockSpec((1, D), lambda i: (0, 0)),
                pl.BlockSpec((1, D), lambda i: (0, 0))]
    args = [xprev, a.reshape(1, D), c.reshape(1, D)]
    for (s, deg, sc, sh) in aggs:
        in_specs.append(pl.BlockSpec((BR, D), lambda i: (i, 0)))
        args.append(s)
        in_specs.append(pl.BlockSpec((BR, 16), lambda i: (i, 0)))
        args.append(deg)
        in_specs.append(pl.BlockSpec((1, D), lambda i: (0, 0)))
        args.append(sc.reshape(1, D))
        in_specs.append(pl.BlockSpec((1, D), lambda i: (0, 0)))
        args.append(sh.reshape(1, D))
    in_specs += [pl.BlockSpec((D, D), lambda i: (0, 0)),
                 pl.BlockSpec((1, D), lambda i: (0, 0)),
                 pl.BlockSpec((D, D), lambda i: (0, 0)),
                 pl.BlockSpec((1, D), lambda i: (0, 0))]
    args += [w1, b1.reshape(1, D), w2, b2.reshape(1, D)]
    outs = [jax.ShapeDtypeStruct((R, D), jnp.float32),
            jax.ShapeDtypeStruct((grid, 8, D), jnp.float32)]
    out_specs = [pl.BlockSpec((BR, D), lambda i: (i, 0)),
                 pl.BlockSpec((1, 8, D), lambda i: (i, 0, 0))]
    return pl.pallas_call(body, grid=(grid,), in_specs=in_specs,
                          out_specs=out_specs, out_shape=outs)(*args)


def _pool_tc(x, a, c, batch3, R, BRP):
    grid = R // BRP

    def body(b_ref, x_ref, a_ref, c_ref, o_ref, cnt_ref):
        i = pl.program_id(0)

        @pl.when(i == 0)
        def _():
            o_ref[...] = jnp.zeros_like(o_ref)
            cnt_ref[...] = jnp.zeros_like(cnt_ref)
        seg = jax.lax.broadcasted_iota(jnp.int32, (NB, BRP), 0)
        onehot = (b_ref[0, 0, :][None, :] == seg).astype(jnp.float32)
        xb = x_ref[...] * a_ref[...] + c_ref[...]
        o_ref[...] += jnp.dot(onehot, xb, preferred_element_type=jnp.float32)
        cnt_ref[...] += jnp.dot(onehot, jnp.ones((BRP, D), jnp.float32),
                                preferred_element_type=jnp.float32)

    return pl.pallas_call(
        body, grid=(grid,),
        in_specs=[pl.BlockSpec((1, 1, BRP), lambda i: (i, 0, 0)),
                  pl.BlockSpec((BRP, D), lambda i: (i, 0)),
                  pl.BlockSpec((1, D), lambda i: (0, 0)),
                  pl.BlockSpec((1, D), lambda i: (0, 0))],
        out_specs=[pl.BlockSpec((NB, D), lambda i: (0, 0)),
                   pl.BlockSpec((NB, D), lambda i: (0, 0))],
        out_shape=[jax.ShapeDtypeStruct((NB, D), jnp.float32),
                   jax.ShapeDtypeStruct((NB, D), jnp.float32)],
    )(batch3, x, a.reshape(1, D), c.reshape(1, D))


def _head_tc(pv, cv, pe, ce, w1, b1, w2, b2):
    CP = 128

    def body(pv_ref, cv_ref, pe_ref, ce_ref, w1_ref, b1_ref, w2_ref,
             b2_ref, o_ref):
        x = pv_ref[...] / cv_ref[...] + pe_ref[...] / ce_ref[...]
        h = jnp.maximum(jnp.dot(x, w1_ref[...],
                                preferred_element_type=jnp.float32)
                        + b1_ref[...], 0.0)
        o_ref[...] = jnp.dot(h, w2_ref[...],
                             preferred_element_type=jnp.float32) + b2_ref[...]

    return pl.pallas_call(
        body, grid=(1,),
        in_specs=[pl.BlockSpec((NB, D), lambda i: (0, 0))] * 4
        + [pl.BlockSpec((D, D), lambda i: (0, 0)),
           pl.BlockSpec((1, D), lambda i: (0, 0)),
           pl.BlockSpec((D, CP), lambda i: (0, 0)),
           pl.BlockSpec((1, CP), lambda i: (0, 0))],
        out_specs=pl.BlockSpec((NB, CP), lambda i: (0, 0)),
        out_shape=jax.ShapeDtypeStruct((NB, CP), jnp.float32),
    )(pv, cv, pe, ce, w1, b1, w2, b2)


_make_filter = functools.cache(_make_filter)
_make_deg = functools.cache(_make_deg)
_make_msgpass = functools.cache(_make_msgpass)

FLUSH_V = 2048
FLUSH_E = 1024


def _msg_stats(st, cr, cp, t_dummy):
    st = st.reshape(NW, 16, 16).astype(jnp.float32)
    s_sum = jnp.sum(st[:, 0:8], axis=0).reshape(D)
    s_sq = jnp.sum(st[:, 8:16], axis=0).reshape(D)
    n_dummy = jnp.sum((cp - cr).reshape(-1, 16)[:, 0]).astype(jnp.float32)
    s_sum = s_sum - n_dummy * t_dummy
    s_sq = s_sq - n_dummy * t_dummy * t_dummy
    mean = s_sum / M
    var = s_sq / M - mean * mean
    return mean, var


def _affine(mean, var, g, be):
    sc = g * jax.lax.rsqrt(var + EPS)
    return sc, be - mean * sc


def kernel(x_v, x_e, v_up_index, v_up_edge, e_down_index, e_down_vert,
           e_up_index, batch_v, batch_e, params):
    p = params
    filt_v, cap_v = _make_filter(3, NP_V, ROWS_V, FLUSH_V)
    filt_e3, cap_e = _make_filter(3, NP_E, SUB_E, FLUSH_E)
    filt_e2, _ = _make_filter(2, NP_E, SUB_E, FLUSH_E)

    vL, vS, vQ, vCR, vCP = filt_v(v_up_index[1], v_up_index[0], v_up_edge)
    dL, dS, dQ, dCR, dCP = filt_e3(e_down_index[1], e_down_index[0],
                                   e_down_vert)
    uL, uS, uCR, uCP = filt_e2(e_up_index[1], e_up_index[0])
    deg_v = _make_deg(NP_V, ROWS_V, cap_v)(vL, vCP)
    deg_v = deg_v.reshape(NW * NP_V * ROWS_V, 16)[:NV]
    deg_d = _make_deg(NP_E, SUB_E, cap_e)(dL, dCP)
    deg_d = deg_d.reshape(NW * NP_E * SUB_E, 16)[:NE]
    deg_u = _make_deg(NP_E, SUB_E, cap_e)(uL, uCP)
    deg_u = deg_u.reshape(NW * NP_E * SUB_E, 16)[:NE]

    vup_pass = _make_msgpass(NP_V, ROWS_V, cap_v, True, True, True)
    ed_pass = _make_msgpass(NP_E, SUB_E, cap_e, True, True, True)
    eu_pass = _make_msgpass(NP_E, SUB_E, cap_e, False, False, False)

    hv, he = x_v, x_e
    a_v = jnp.ones((D,), jnp.float32)
    c_v = jnp.zeros((D,), jnp.float32)
    a_e = jnp.ones((D,), jnp.float32)
    c_e = jnp.zeros((D,), jnp.float32)

    for l in range(3):
        wv = p[f"L{l}_vup_W"]
        we = p[f"L{l}_edown_W"]
        wu = p[f"L{l}_eup_W"]
        wv_cat = jnp.concatenate([wv[:D], we[D:]], axis=1)
        wv_f = a_v[:, None] * wv_cat
        bv_f = c_v @ wv_cat
        Pv, Qv = _tables_tc(hv, wv_f, bv_f, 0)
        we_cat = jnp.concatenate([wv[D:], we[:D], wu], axis=1)
        we_f = a_e[:, None] * we_cat
        be_f = c_e @ we_cat
        be_f = be_f.at[2 * D:].add(p[f"L{l}_eup_b"])
        Qe, Pe, Ru = _tables_tc(he, we_f, be_f, D)

        sv_flat, v_st = vup_pass(Pv, Qe, p[f"L{l}_vup_b"], vL, vS, vQ, vCP)
        Sv = sv_flat.reshape(NW * NP_V * ROWS_V, D)[:NV]
        se_flat, d_st = ed_pass(Pe, Qv, p[f"L{l}_edown_b"], dL, dS, dQ, dCP)
        Se = se_flat.reshape(NW * NP_E * SUB_E, D)[:NE]
        su_flat, u_st = eu_pass(Ru, p[f"L{l}_eup_b"], uL, uS, uCP)
        Su = su_flat.reshape(NW * NP_E * SUB_E, D)[:NE]

        td_v = jnp.maximum(Pv[0] + Qe[0] + p[f"L{l}_vup_b"], 0.0)
        mean, var = _msg_stats(v_st, vCR, vCP, td_v)
        sc_v, sh_v = _affine(mean, var, p[f"L{l}_vup_g"], p[f"L{l}_vup_be"])
        td_d = jnp.maximum(Pe[0] + Qv[0] + p[f"L{l}_edown_b"], 0.0)
        mean, var = _msg_stats(d_st, dCR, dCP, td_d)
        sc_d, sh_d = _affine(mean, var, p[f"L{l}_edown_g"],
                             p[f"L{l}_edown_be"])
        td_u = Ru[0]
        mean, var = _msg_stats(u_st, uCR, uCP, td_u)
        sc_u, sh_u = _affine(mean, var, p[f"L{l}_eup_g"], p[f"L{l}_eup_be"])

        hv, v_ps = _update_tc(hv, a_v, c_v,
                              [(Sv, deg_v, sc_v, sh_v)],
                              p[f"L{l}_vupd_W1"], p[f"L{l}_vupd_b1"],
                              p[f"L{l}_vupd_W2"], p[f"L{l}_vupd_b2"])
        he, e_ps = _update_tc(he, a_e, c_e,
                              [(Se, deg_d, sc_d, sh_d),
                               (Su, deg_u, sc_u, sh_u)],
                              p[f"L{l}_eupd_W1"], p[f"L{l}_eupd_b1"],
                              p[f"L{l}_eupd_W2"], p[f"L{l}_eupd_b2"])
        s1 = jnp.sum(v_ps[:, 0], axis=0)
        s2 = jnp.sum(v_ps[:, 1], axis=0)
        mean = s1 / NV
        var = s2 / NV - mean * mean
        a_v, c_v = _affine(mean, var, p[f"L{l}_vupd_g"], p[f"L{l}_vupd_be"])
        s1 = jnp.sum(e_ps[:, 0], axis=0)
        s2 = jnp.sum(e_ps[:, 1], axis=0)
        mean = s1 / NE
        var = s2 / NE - mean * mean
        a_e, c_e = _affine(mean, var, p[f"L{l}_eupd_g"], p[f"L{l}_eupd_be"])

    bv3 = batch_v.reshape(5, 1, 2000)
    be3 = batch_e.reshape(80, 1, 2000)
    pv, cv = _pool_tc(hv, a_v, c_v, bv3, NV, 2000)
    pe, ce = _pool_tc(he, a_e, c_e, be3, NE, 2000)
    cv = jnp.maximum(cv, 1.0)
    ce = jnp.maximum(ce, 1.0)
    w2p = jnp.zeros((D, 128), jnp.float32).at[:, :10].set(p["lin2_W"])
    b2p = jnp.zeros((128,), jnp.float32).at[:10].set(p["lin2_b"])
    out = _head_tc(pv, cv, pe, ce, p["lin1_W"], p["lin1_b"].reshape(1, D),
                   w2p, b2p.reshape(1, 128))
    return out[:, :10]
